# Initial kernel scaffold; baseline (speedup 1.0000x reference)
import jax
import jax.numpy as jnp
from jax.experimental import pallas as pl


def kernel(edge_index, emb, W0, a_src0, a_dst0, b0, W1, a_src1, a_dst1, b1, W2, a_src2, a_dst2, b2):
    raise NotImplementedError("write your pallas kernel here")


# speedup vs baseline: 13.3303x; 13.3303x over previous
import dataclasses

import jax
import jax.numpy as jnp
from jax import lax
from jax.experimental import pallas as pl
from jax.experimental.pallas import tpu as pltpu
from jax.experimental.pallas import tpu_sc as plsc

N = 50000
D = 64
E = 800000

LANES = 16
NC = 2
NS = 16
HALF = D // NC

ROWBLK = 256
NPAD = 50176
NBLK = NPAD // ROWBLK

CHUNK = 128
EPAD = 819200
CHUNKS_PER_SUB = EPAD // (CHUNK * NS)
NPADROWS = 48
DENROWS = NPAD // LANES
NROWS = NPAD * 2 + DENROWS


def _prep_body(x_ref, w_ref, asr_ref, adr_ref,
               h_ref, acc_ref, al_ref, ar_ref, den_ref):
    x = x_ref[...]
    h = jnp.dot(x, w_ref[...], preferred_element_type=jnp.float32)
    al = jnp.sum(h * asr_ref[...], axis=1)
    ar = jnp.sum(h * adr_ref[...], axis=1)
    e = al + ar
    e = jnp.where(e >= 0.0, e, e * 0.2)
    exs = jnp.exp(e)
    h_ref[0, ...] = h[:, :HALF]
    h_ref[1, ...] = h[:, HALF:]
    acc = h * exs[:, None]
    acc_ref[0, ...] = acc[:, :HALF]
    acc_ref[1, ...] = acc[:, HALF:]
    al_ref[...] = al
    ar_ref[...] = ar
    den_ref[...] = exs


def _prep(x, w, a_s, a_d):
    return pl.pallas_call(
        _prep_body,
        grid=(NBLK,),
        in_specs=[
            pl.BlockSpec((ROWBLK, D), lambda i: (i, 0)),
            pl.BlockSpec((D, D), lambda i: (0, 0)),
            pl.BlockSpec((D,), lambda i: (0,)),
            pl.BlockSpec((D,), lambda i: (0,)),
        ],
        out_specs=[
            pl.BlockSpec((2, ROWBLK, HALF), lambda i: (0, i, 0)),
            pl.BlockSpec((2, ROWBLK, HALF), lambda i: (0, i, 0)),
            pl.BlockSpec((ROWBLK,), lambda i: (i,)),
            pl.BlockSpec((ROWBLK,), lambda i: (i,)),
            pl.BlockSpec((ROWBLK,), lambda i: (i,)),
        ],
        out_shape=[
            jax.ShapeDtypeStruct((2, NPAD, HALF), jnp.float32),
            jax.ShapeDtypeStruct((2, NPAD, HALF), jnp.float32),
            jax.ShapeDtypeStruct((NPAD,), jnp.float32),
            jax.ShapeDtypeStruct((NPAD,), jnp.float32),
            jax.ShapeDtypeStruct((NPAD,), jnp.float32),
        ],
    )(x, w, a_s, a_d)


def _fin_body(acc_ref, den_ref, b_ref, x_ref):
    a0 = acc_ref[0]
    a1 = acc_ref[1]
    acc = jnp.concatenate([a0, a1], axis=1)
    r = 1.0 / (den_ref[...] + 1e-16)
    o = acc * r[:, None] + b_ref[...]
    x_ref[...] = jnp.where(o > 0.0, o, jnp.exp(o) - 1.0)


def _fin(acc, den, b):
    return pl.pallas_call(
        _fin_body,
        grid=(NBLK,),
        in_specs=[
            pl.BlockSpec((2, ROWBLK, HALF), lambda i: (0, i, 0)),
            pl.BlockSpec((ROWBLK,), lambda i: (i,)),
            pl.BlockSpec((D,), lambda i: (0,)),
        ],
        out_specs=pl.BlockSpec((ROWBLK, D), lambda i: (i, 0)),
        out_shape=jax.ShapeDtypeStruct((NPAD, D), jnp.float32),
    )(acc, den, b)


def _fin_mean_body(acc_ref, den_ref, b_ref, e_ref, x1_ref, x2_ref, o_ref):
    acc = jnp.concatenate([acc_ref[0], acc_ref[1]], axis=1)
    r = 1.0 / (den_ref[...] + 1e-16)
    o = acc * r[:, None] + b_ref[...]
    x3 = jnp.where(o > 0.0, o, jnp.exp(o) - 1.0)
    o_ref[...] = (e_ref[...] + x1_ref[...] + x2_ref[...] + x3) * 0.25


def _fin_mean(acc, den, b, emb, x1, x2):
    return pl.pallas_call(
        _fin_mean_body,
        grid=(NBLK,),
        in_specs=[
            pl.BlockSpec((2, ROWBLK, HALF), lambda i: (0, i, 0)),
            pl.BlockSpec((ROWBLK,), lambda i: (i,)),
            pl.BlockSpec((D,), lambda i: (0,)),
            pl.BlockSpec((ROWBLK, D), lambda i: (i, 0)),
            pl.BlockSpec((ROWBLK, D), lambda i: (i, 0)),
            pl.BlockSpec((ROWBLK, D), lambda i: (i, 0)),
        ],
        out_specs=pl.BlockSpec((ROWBLK, D), lambda i: (i, 0)),
        out_shape=jax.ShapeDtypeStruct((NPAD, D), jnp.float32),
    )(acc, den, b, emb, x1, x2)


def _edge_body(src_hbm, dst_hbm, h2_hbm, al_hbm, ar_hbm, acc0_hbm, den0_hbm,
               acco_hbm,
               acc_sh, al_sh, ar_sh, tmp_v, rows_v,
               srcv, dstv, alv, arv, exv, ia_v, ib_v, da_v, db_v, drv, oh_v):
    c = lax.axis_index("c")
    s = lax.axis_index("s")
    acc_rows_per_sub = (NPAD * 2) // NS
    den_rows_per_sub = DENROWS // 8
    @pl.loop(0, 8)
    def _init(j):
        off = s * acc_rows_per_sub + j * (acc_rows_per_sub // 8)
        pltpu.sync_copy(acc0_hbm.at[c, pl.ds(off, acc_rows_per_sub // 8)],
                        acc_sh.at[pl.ds(off, acc_rows_per_sub // 8)])

    @pl.when(s < 8)
    def _():
        pltpu.sync_copy(den0_hbm.at[pl.ds(s * den_rows_per_sub, den_rows_per_sub)],
                        acc_sh.at[pl.ds(NPAD * 2 + s * den_rows_per_sub,
                                        den_rows_per_sub)])
    tab_per_sub = NPAD // NS
    pltpu.sync_copy(al_hbm.at[pl.ds(s * tab_per_sub, tab_per_sub)], tmp_v)
    pltpu.sync_copy(tmp_v, al_sh.at[pl.ds(s * tab_per_sub, tab_per_sub)])
    pltpu.sync_copy(ar_hbm.at[pl.ds(s * tab_per_sub, tab_per_sub)], tmp_v)
    pltpu.sync_copy(tmp_v, ar_sh.at[pl.ds(s * tab_per_sub, tab_per_sub)])
    plsc.subcore_barrier()

    coff = c * (NPAD * 2)
    base0 = s * (CHUNKS_PER_SUB * CHUNK)

    @pl.loop(0, CHUNKS_PER_SUB)
    def _chunk(t):
        base = base0 + t * CHUNK
        pltpu.sync_copy(src_hbm.at[pl.ds(base, CHUNK)], srcv)
        pltpu.sync_copy(dst_hbm.at[pl.ds(base, CHUNK)], dstv)
        pltpu.sync_copy(al_sh.at[srcv], alv)
        pltpu.sync_copy(ar_sh.at[dstv], arv)
        zero16 = jnp.zeros((LANES,), jnp.float32)
        for k in range(CHUNK // LANES):
            sl = pl.ds(k * LANES, LANES)
            s16 = srcv[sl]
            d16 = dstv[sl]
            av = alv[sl]
            bv = arv[sl]
            e = av + bv
            e = jnp.where(e >= 0.0, e, e * 0.2)
            ex = jnp.exp(e)
            exv[sl] = ex
            ia = coff + 2 * s16
            ia_v[sl] = ia
            ib_v[sl] = ia + 1
            da = 2 * d16
            da_v[sl] = da
            db_v[sl] = da + 1
            drv[sl] = NPAD * 2 + lax.shift_right_logical(d16, 4)
            for j in range(LANES):
                oh_v.at[k * LANES + j][...] = zero16
            rowid = k * LANES + lax.iota(jnp.int32, LANES)
            plsc.store_scatter(oh_v, [rowid, d16 & 15], ex)
        pltpu.sync_copy(h2_hbm.at[ia_v], rows_v.at[pl.ds(0, CHUNK)])
        pltpu.sync_copy(h2_hbm.at[ib_v], rows_v.at[pl.ds(CHUNK, CHUNK)])

        @pl.loop(0, CHUNK)
        def _scale(r):
            bc = plsc.load_gather(exv, [jnp.full((LANES,), r, jnp.int32)])
            rows_v.at[r][...] = rows_v.at[r][...] * bc
            rows_v.at[CHUNK + r][...] = rows_v.at[CHUNK + r][...] * bc

        pltpu.sync_copy(rows_v.at[pl.ds(0, CHUNK)], acc_sh.at[da_v], add=True)
        pltpu.sync_copy(rows_v.at[pl.ds(CHUNK, CHUNK)], acc_sh.at[db_v],
                        add=True)
        pltpu.sync_copy(oh_v, acc_sh.at[drv], add=True)

    plsc.subcore_barrier()

    @pl.loop(0, 8)
    def _exp(j):
        off = s * acc_rows_per_sub + j * (acc_rows_per_sub // 8)
        pltpu.sync_copy(acc_sh.at[pl.ds(off, acc_rows_per_sub // 8)],
                        acco_hbm.at[c, pl.ds(off, acc_rows_per_sub // 8)])

    @pl.when(s < 8)
    def _():
        pltpu.sync_copy(acc_sh.at[pl.ds(NPAD * 2 + s * den_rows_per_sub,
                                        den_rows_per_sub)],
                        acco_hbm.at[c, pl.ds(NPAD * 2 + s * den_rows_per_sub,
                                             den_rows_per_sub)])


def _edge(src, dst, h2, al, ar, acc0, den0):
    mesh = plsc.VectorSubcoreMesh(core_axis_name="c", subcore_axis_name="s",
                                  num_cores=NC, num_subcores=NS)
    cp = pltpu.CompilerParams(use_tc_tiling_on_sc=False)
    if "needs_layout_passes" in pltpu.CompilerParams.__dataclass_fields__:
        cp = dataclasses.replace(cp, needs_layout_passes=False)
    f = pl.kernel(
        _edge_body,
        out_type=[
            jax.ShapeDtypeStruct((2, NROWS, LANES), jnp.float32),
        ],
        mesh=mesh,
        scratch_types=[
            pltpu.VMEM_SHARED((NROWS, LANES), jnp.float32),
            pltpu.VMEM_SHARED((NPAD,), jnp.float32),
            pltpu.VMEM_SHARED((NPAD,), jnp.float32),
            pltpu.VMEM((NPAD // NS,), jnp.float32),
            pltpu.VMEM((2 * CHUNK, LANES), jnp.float32),
            pltpu.VMEM((CHUNK,), jnp.int32),
            pltpu.VMEM((CHUNK,), jnp.int32),
            pltpu.VMEM((CHUNK,), jnp.float32),
            pltpu.VMEM((CHUNK,), jnp.float32),
            pltpu.VMEM((CHUNK,), jnp.float32),
            pltpu.VMEM((CHUNK,), jnp.int32),
            pltpu.VMEM((CHUNK,), jnp.int32),
            pltpu.VMEM((CHUNK,), jnp.int32),
            pltpu.VMEM((CHUNK,), jnp.int32),
            pltpu.VMEM((CHUNK,), jnp.int32),
            pltpu.VMEM((CHUNK, LANES), jnp.float32),
        ],
        compiler_params=cp,
    )
    return f(src, dst, h2, al, ar, acc0, den0)


def kernel(edge_index, emb, W0, a_src0, a_dst0, b0,
           W1, a_src1, a_dst1, b1, W2, a_src2, a_dst2, b2):
    ei = edge_index.astype(jnp.int32)
    pad_nodes = N + (jnp.arange(EPAD - E, dtype=jnp.int32) % NPADROWS)
    src = jnp.concatenate([ei[0], pad_nodes])
    dst = jnp.concatenate([ei[1], pad_nodes])

    emb_pad = jnp.zeros((NPAD, D), jnp.float32).at[:N].set(emb)

    x = emb_pad
    hist = []
    out = None
    for li, (w, a_s, a_d, b) in enumerate(
            ((W0, a_src0, a_dst0, b0), (W1, a_src1, a_dst1, b1),
             (W2, a_src2, a_dst2, b2))):
        h_all, acc_all, al, ar, den0 = _prep(x, w, a_s, a_d)
        h2 = h_all.reshape(2 * NPAD * 2, LANES)
        acc0 = acc_all.reshape(2, NPAD * 2, LANES)
        (acco,) = _edge(src, dst, h2, al, ar, acc0,
                        den0.reshape(DENROWS, LANES))
        accr = acco[:, :NPAD * 2, :].reshape(2, NPAD, HALF)
        den = acco[0, NPAD * 2:, :].reshape(NPAD)
        if li < 2:
            x = _fin(accr, den, b)
            hist.append(x)
        else:
            out = _fin_mean(accr, den, b, emb_pad, hist[0], hist[1])
    return out[:N]


# speedup vs baseline: 19.7001x; 1.4778x over previous
import dataclasses

import jax
import jax.numpy as jnp
from jax import lax
from jax.experimental import pallas as pl
from jax.experimental.pallas import tpu as pltpu
from jax.experimental.pallas import tpu_sc as plsc

N = 50000
D = 64
E = 800000

LANES = 16
NC = 2
NS = 16
HALF = D // NC

ROWBLK = 256
NPAD = 50176
NBLK = NPAD // ROWBLK

CHUNK = 128
EPAD = 819200
CHUNKS_PER_SUB = EPAD // (CHUNK * NS)
NPADROWS = 48
DENROWS = NPAD // LANES
NROWS = NPAD * 2 + DENROWS


def _prep_body(x_ref, w_ref, asr_ref, adr_ref,
               h_ref, acc_ref, al_ref, ar_ref, den_ref):
    x = x_ref[...]
    h = jnp.dot(x, w_ref[...], preferred_element_type=jnp.float32)
    al = jnp.sum(h * asr_ref[...], axis=1)
    ar = jnp.sum(h * adr_ref[...], axis=1)
    e = al + ar
    e = jnp.where(e >= 0.0, e, e * 0.2)
    exs = jnp.exp(e)
    h_ref[0, ...] = h[:, :HALF]
    h_ref[1, ...] = h[:, HALF:]
    acc = h * exs[:, None]
    acc_ref[0, ...] = acc[:, :HALF]
    acc_ref[1, ...] = acc[:, HALF:]
    al_ref[...] = al
    ar_ref[...] = ar
    den_ref[...] = exs


def _prep(x, w, a_s, a_d):
    return pl.pallas_call(
        _prep_body,
        grid=(NBLK,),
        in_specs=[
            pl.BlockSpec((ROWBLK, D), lambda i: (i, 0)),
            pl.BlockSpec((D, D), lambda i: (0, 0)),
            pl.BlockSpec((D,), lambda i: (0,)),
            pl.BlockSpec((D,), lambda i: (0,)),
        ],
        out_specs=[
            pl.BlockSpec((2, ROWBLK, HALF), lambda i: (0, i, 0)),
            pl.BlockSpec((2, ROWBLK, HALF), lambda i: (0, i, 0)),
            pl.BlockSpec((ROWBLK,), lambda i: (i,)),
            pl.BlockSpec((ROWBLK,), lambda i: (i,)),
            pl.BlockSpec((ROWBLK,), lambda i: (i,)),
        ],
        out_shape=[
            jax.ShapeDtypeStruct((2, NPAD, HALF), jnp.float32),
            jax.ShapeDtypeStruct((2, NPAD, HALF), jnp.float32),
            jax.ShapeDtypeStruct((NPAD,), jnp.float32),
            jax.ShapeDtypeStruct((NPAD,), jnp.float32),
            jax.ShapeDtypeStruct((NPAD,), jnp.float32),
        ],
    )(x, w, a_s, a_d)


def _fin_body(acc_ref, den_ref, b_ref, x_ref):
    a0 = acc_ref[0]
    a1 = acc_ref[1]
    acc = jnp.concatenate([a0, a1], axis=1)
    r = 1.0 / (den_ref[...] + 1e-16)
    o = acc * r[:, None] + b_ref[...]
    x_ref[...] = jnp.where(o > 0.0, o, jnp.exp(o) - 1.0)


def _fin(acc, den, b):
    return pl.pallas_call(
        _fin_body,
        grid=(NBLK,),
        in_specs=[
            pl.BlockSpec((2, ROWBLK, HALF), lambda i: (0, i, 0)),
            pl.BlockSpec((ROWBLK,), lambda i: (i,)),
            pl.BlockSpec((D,), lambda i: (0,)),
        ],
        out_specs=pl.BlockSpec((ROWBLK, D), lambda i: (i, 0)),
        out_shape=jax.ShapeDtypeStruct((NPAD, D), jnp.float32),
    )(acc, den, b)


def _fin_mean_body(acc_ref, den_ref, b_ref, e_ref, x1_ref, x2_ref, o_ref):
    acc = jnp.concatenate([acc_ref[0], acc_ref[1]], axis=1)
    r = 1.0 / (den_ref[...] + 1e-16)
    o = acc * r[:, None] + b_ref[...]
    x3 = jnp.where(o > 0.0, o, jnp.exp(o) - 1.0)
    o_ref[...] = (e_ref[...] + x1_ref[...] + x2_ref[...] + x3) * 0.25


def _fin_mean(acc, den, b, emb, x1, x2):
    return pl.pallas_call(
        _fin_mean_body,
        grid=(NBLK,),
        in_specs=[
            pl.BlockSpec((2, ROWBLK, HALF), lambda i: (0, i, 0)),
            pl.BlockSpec((ROWBLK,), lambda i: (i,)),
            pl.BlockSpec((D,), lambda i: (0,)),
            pl.BlockSpec((ROWBLK, D), lambda i: (i, 0)),
            pl.BlockSpec((ROWBLK, D), lambda i: (i, 0)),
            pl.BlockSpec((ROWBLK, D), lambda i: (i, 0)),
        ],
        out_specs=pl.BlockSpec((ROWBLK, D), lambda i: (i, 0)),
        out_shape=jax.ShapeDtypeStruct((NPAD, D), jnp.float32),
    )(acc, den, b, emb, x1, x2)


def _edge_body(ed_hbm, h2_hbm, al_hbm, ar_hbm, acc0_hbm, den0_hbm,
               acco_hbm,
               acc_sh, alr_sh, tmp_v,
               rows, oh, sd, alv, arv, exv, ia, ib, da, db, dr,
               gs, ss, es, asm):
    c = lax.axis_index("c")
    s = lax.axis_index("s")
    acc_rows_per_sub = (NPAD * 2) // NS
    den_rows_per_sub = DENROWS // 8
    tab_per_sub = NPAD // NS

    @pl.loop(0, 56)
    def _init(j):
        off = s * acc_rows_per_sub + j * 112
        pltpu.sync_copy(acc0_hbm.at[c, pl.ds(off, 112)],
                        acc_sh.at[pl.ds(off, 112)])

    @pl.when(s < 8)
    def _():
        @pl.loop(0, 7)
        def _initd(j):
            off = s * den_rows_per_sub + j * 56
            pltpu.sync_copy(den0_hbm.at[pl.ds(off, 56)],
                            acc_sh.at[pl.ds(NPAD * 2 + off, 56)])

    for part, hbm in ((0, al_hbm), (1, ar_hbm)):
        @pl.loop(0, 8)
        def _tab(j, part=part, hbm=hbm):
            off = s * tab_per_sub + j * 392
            pltpu.sync_copy(hbm.at[pl.ds(off, 392)], tmp_v)
            pltpu.sync_copy(tmp_v, alr_sh.at[pl.ds(part * NPAD + off, 392)])
    plsc.subcore_barrier()

    coff = c * (NPAD * 2)
    base0 = s * CHUNKS_PER_SUB

    def load_ed(t, b):
        return pltpu.async_copy(
            ed_hbm.at[pl.ds((base0 + t) * (2 * CHUNK), 2 * CHUNK)], sd[b],
            es[b])

    def wait_ed(t, b):
        pltpu.make_async_copy(
            ed_hbm.at[pl.ds((base0 + t) * (2 * CHUNK), 2 * CHUNK)], sd[b],
            es[b]).wait()

    def issue_alr(b):
        pltpu.async_copy(alr_sh.at[sd[b].at[pl.ds(0, CHUNK)]], alv[b], asm[b])
        pltpu.async_copy(alr_sh.at[sd[b].at[pl.ds(CHUNK, CHUNK)]], arv[b],
                         asm[b])

    def wait_alr(b):
        pltpu.make_async_copy(alr_sh.at[sd[b].at[pl.ds(0, CHUNK)]], alv[b],
                              asm[b]).wait()
        pltpu.make_async_copy(alr_sh.at[sd[b].at[pl.ds(CHUNK, CHUNK)]],
                              arv[b], asm[b]).wait()

    def compute_chunk(b):
        zero16 = jnp.zeros((LANES,), jnp.float32)
        for k in range(CHUNK // LANES):
            sl = pl.ds(k * LANES, LANES)
            s16 = sd[b][sl]
            dn16 = sd[b][pl.ds(CHUNK + k * LANES, LANES)]
            e = alv[b][sl] + arv[b][sl]
            e = jnp.where(e >= 0.0, e, e * 0.2)
            ex = jnp.exp(e)
            exv[b][sl] = ex
            d16 = dn16 - NPAD
            iav = coff + 2 * s16
            ia[b][sl] = iav
            ib[b][sl] = iav + 1
            dav = 2 * d16
            da[b][sl] = dav
            db[b][sl] = dav + 1
            dr[b][sl] = NPAD * 2 + lax.shift_right_logical(d16, 4)
            for j in range(LANES):
                oh[b].at[k * LANES + j][...] = zero16
            rowid = k * LANES + lax.iota(jnp.int32, LANES)
            plsc.store_scatter(oh[b], [rowid, d16 & 15], ex)

    def issue_gather(b):
        pltpu.async_copy(h2_hbm.at[ia[b]], rows[b].at[pl.ds(0, CHUNK)], gs[b])
        pltpu.async_copy(h2_hbm.at[ib[b]], rows[b].at[pl.ds(CHUNK, CHUNK)],
                         gs[b])

    def wait_gather(b):
        pltpu.make_async_copy(h2_hbm.at[ia[b]], rows[b].at[pl.ds(0, CHUNK)],
                              gs[b]).wait()
        pltpu.make_async_copy(h2_hbm.at[ib[b]],
                              rows[b].at[pl.ds(CHUNK, CHUNK)], gs[b]).wait()

    def scale_chunk(b):
        @pl.loop(0, CHUNK)
        def _scale(r):
            bc = plsc.load_gather(exv[b], [jnp.full((LANES,), r, jnp.int32)])
            rows[b].at[r][...] = rows[b].at[r][...] * bc
            rows[b].at[CHUNK + r][...] = rows[b].at[CHUNK + r][...] * bc

    def issue_scatter(b):
        pltpu.async_copy(rows[b].at[pl.ds(0, CHUNK)], acc_sh.at[da[b]], ss[b],
                         add=True)
        pltpu.async_copy(rows[b].at[pl.ds(CHUNK, CHUNK)], acc_sh.at[db[b]],
                         ss[b], add=True)
        pltpu.async_copy(oh[b], acc_sh.at[dr[b]], ss[b], add=True)

    def wait_scatter(b):
        pltpu.make_async_copy(rows[b].at[pl.ds(0, CHUNK)], acc_sh.at[da[b]],
                              ss[b]).wait()
        pltpu.make_async_copy(rows[b].at[pl.ds(CHUNK, CHUNK)],
                              acc_sh.at[db[b]], ss[b]).wait()
        pltpu.make_async_copy(oh[b], acc_sh.at[dr[b]], ss[b]).wait()

    pltpu.sync_copy(ed_hbm.at[pl.ds(base0 * (2 * CHUNK), 2 * CHUNK)], sd[0])
    pltpu.sync_copy(alr_sh.at[sd[0].at[pl.ds(0, CHUNK)]], alv[0])
    pltpu.sync_copy(alr_sh.at[sd[0].at[pl.ds(CHUNK, CHUNK)]], arv[0])
    compute_chunk(0)
    issue_gather(0)
    load_ed(1, 1)

    @pl.loop(0, CHUNKS_PER_SUB // 2)
    def _g(i):
        for b in (0, 1):
            o = 1 - b
            t = 2 * i + b
            wait_gather(b)
            scale_chunk(b)
            issue_scatter(b)
            tn = t + 1

            @pl.when(tn < CHUNKS_PER_SUB)
            def _():
                wait_ed(tn, o)
                issue_alr(o)

                @pl.when(tn + 1 < CHUNKS_PER_SUB)
                def _():
                    load_ed(tn + 1, b)

                @pl.when(t >= 1)
                def _():
                    wait_scatter(o)
                wait_alr(o)
                compute_chunk(o)
                issue_gather(o)

    wait_scatter(0)
    wait_scatter(1)
    plsc.subcore_barrier()

    @pl.loop(0, 56)
    def _exp(j):
        off = s * acc_rows_per_sub + j * 112
        pltpu.sync_copy(acc_sh.at[pl.ds(off, 112)],
                        acco_hbm.at[c, pl.ds(off, 112)])

    @pl.when(s < 8)
    def _():
        @pl.loop(0, 7)
        def _expd(j):
            off = s * den_rows_per_sub + j * 56
            pltpu.sync_copy(acc_sh.at[pl.ds(NPAD * 2 + off, 56)],
                            acco_hbm.at[c, pl.ds(NPAD * 2 + off, 56)])


def _edge(ed, h2, al, ar, acc0, den0):
    mesh = plsc.VectorSubcoreMesh(core_axis_name="c", subcore_axis_name="s",
                                  num_cores=NC, num_subcores=NS)
    cp = pltpu.CompilerParams(use_tc_tiling_on_sc=False)
    if "needs_layout_passes" in pltpu.CompilerParams.__dataclass_fields__:
        cp = dataclasses.replace(cp, needs_layout_passes=False)
    f = pl.kernel(
        _edge_body,
        out_type=[
            jax.ShapeDtypeStruct((2, NROWS, LANES), jnp.float32),
        ],
        mesh=mesh,
        scratch_types=[
            pltpu.VMEM_SHARED((NROWS, LANES), jnp.float32),
            pltpu.VMEM_SHARED((2 * NPAD,), jnp.float32),
            pltpu.VMEM((392,), jnp.float32),
            [pltpu.VMEM((2 * CHUNK, LANES), jnp.float32) for _ in range(2)],
            [pltpu.VMEM((CHUNK, LANES), jnp.float32) for _ in range(2)],
            [pltpu.VMEM((2 * CHUNK,), jnp.int32) for _ in range(2)],
            [pltpu.VMEM((CHUNK,), jnp.float32) for _ in range(2)],
            [pltpu.VMEM((CHUNK,), jnp.float32) for _ in range(2)],
            [pltpu.VMEM((CHUNK,), jnp.float32) for _ in range(2)],
            [pltpu.VMEM((CHUNK,), jnp.int32) for _ in range(2)],
            [pltpu.VMEM((CHUNK,), jnp.int32) for _ in range(2)],
            [pltpu.VMEM((CHUNK,), jnp.int32) for _ in range(2)],
            [pltpu.VMEM((CHUNK,), jnp.int32) for _ in range(2)],
            [pltpu.VMEM((CHUNK,), jnp.int32) for _ in range(2)],
            [pltpu.SemaphoreType.DMA for _ in range(2)],
            [pltpu.SemaphoreType.DMA for _ in range(2)],
            [pltpu.SemaphoreType.DMA for _ in range(2)],
            [pltpu.SemaphoreType.DMA for _ in range(2)],
        ],
        compiler_params=cp,
    )
    return f(ed, h2, al, ar, acc0, den0)


def kernel(edge_index, emb, W0, a_src0, a_dst0, b0,
           W1, a_src1, a_dst1, b1, W2, a_src2, a_dst2, b2):
    ei = edge_index.astype(jnp.int32)
    pad_nodes = N + (jnp.arange(EPAD - E, dtype=jnp.int32) % NPADROWS)
    srci = jnp.concatenate([ei[0], pad_nodes])
    dsti = jnp.concatenate([ei[1], pad_nodes])
    ed = jnp.stack([srci.reshape(-1, CHUNK),
                    (dsti + NPAD).reshape(-1, CHUNK)], axis=1).reshape(-1)

    emb_pad = jnp.zeros((NPAD, D), jnp.float32).at[:N].set(emb)

    x = emb_pad
    hist = []
    out = None
    for li, (w, a_s, a_d, b) in enumerate(
            ((W0, a_src0, a_dst0, b0), (W1, a_src1, a_dst1, b1),
             (W2, a_src2, a_dst2, b2))):
        h_all, acc_all, al, ar, den0 = _prep(x, w, a_s, a_d)
        h2 = h_all.reshape(2 * NPAD * 2, LANES)
        acc0 = acc_all.reshape(2, NPAD * 2, LANES)
        (acco,) = _edge(ed, h2, al, ar, acc0,
                        den0.reshape(DENROWS, LANES))
        accr = acco[:, :NPAD * 2, :].reshape(2, NPAD, HALF)
        den = acco[0, NPAD * 2:, :].reshape(NPAD)
        if li < 2:
            x = _fin(accr, den, b)
            hist.append(x)
        else:
            out = _fin_mean(accr, den, b, emb_pad, hist[0], hist[1])
    return out[:N]


# speedup vs baseline: 24.6251x; 1.2500x over previous
import dataclasses

import jax
import jax.numpy as jnp
from jax import lax
from jax.experimental import pallas as pl
from jax.experimental.pallas import tpu as pltpu
from jax.experimental.pallas import tpu_sc as plsc

N = 50000
D = 64
E = 800000

LANES = 16
NC = 2
NS = 16
HALF = D // NC

ROWBLK = 256
NPAD = 50176
NBLK = NPAD // ROWBLK

CHUNK = 128
EPAD = 819200
CHUNKS_PER_SUB = EPAD // (CHUNK * NS)
NPADROWS = 48
DENROWS = NPAD // LANES
NROWS = NPAD * 2 + DENROWS


def _prep_body(x_ref, w_ref, asr_ref, adr_ref,
               h_ref, acc_ref, al_ref, ar_ref, den_ref):
    x = x_ref[...]
    h = jnp.dot(x, w_ref[...], preferred_element_type=jnp.float32)
    al = jnp.sum(h * asr_ref[...], axis=1)
    ar = jnp.sum(h * adr_ref[...], axis=1)
    e = al + ar
    e = jnp.where(e >= 0.0, e, e * 0.2)
    exs = jnp.exp(e)
    h_ref[0, ...] = h[:, :HALF]
    h_ref[1, ...] = h[:, HALF:]
    acc = h * exs[:, None]
    acc_ref[0, ...] = acc[:, :HALF]
    acc_ref[1, ...] = acc[:, HALF:]
    al_ref[...] = al
    ar_ref[...] = ar
    den_ref[...] = exs


def _prep(x, w, a_s, a_d):
    return pl.pallas_call(
        _prep_body,
        grid=(NBLK,),
        in_specs=[
            pl.BlockSpec((ROWBLK, D), lambda i: (i, 0)),
            pl.BlockSpec((D, D), lambda i: (0, 0)),
            pl.BlockSpec((D,), lambda i: (0,)),
            pl.BlockSpec((D,), lambda i: (0,)),
        ],
        out_specs=[
            pl.BlockSpec((2, ROWBLK, HALF), lambda i: (0, i, 0)),
            pl.BlockSpec((2, ROWBLK, HALF), lambda i: (0, i, 0)),
            pl.BlockSpec((ROWBLK,), lambda i: (i,)),
            pl.BlockSpec((ROWBLK,), lambda i: (i,)),
            pl.BlockSpec((ROWBLK,), lambda i: (i,)),
        ],
        out_shape=[
            jax.ShapeDtypeStruct((2, NPAD, HALF), jnp.float32),
            jax.ShapeDtypeStruct((2, NPAD, HALF), jnp.float32),
            jax.ShapeDtypeStruct((NPAD,), jnp.float32),
            jax.ShapeDtypeStruct((NPAD,), jnp.float32),
            jax.ShapeDtypeStruct((NPAD,), jnp.float32),
        ],
    )(x, w, a_s, a_d)


def _fin_body(acc_ref, den_ref, b_ref, x_ref):
    a0 = acc_ref[0]
    a1 = acc_ref[1]
    acc = jnp.concatenate([a0, a1], axis=1)
    r = 1.0 / (den_ref[...] + 1e-16)
    o = acc * r[:, None] + b_ref[...]
    x_ref[...] = jnp.where(o > 0.0, o, jnp.exp(o) - 1.0)


def _fin(acc, den, b):
    return pl.pallas_call(
        _fin_body,
        grid=(NBLK,),
        in_specs=[
            pl.BlockSpec((2, ROWBLK, HALF), lambda i: (0, i, 0)),
            pl.BlockSpec((ROWBLK,), lambda i: (i,)),
            pl.BlockSpec((D,), lambda i: (0,)),
        ],
        out_specs=pl.BlockSpec((ROWBLK, D), lambda i: (i, 0)),
        out_shape=jax.ShapeDtypeStruct((NPAD, D), jnp.float32),
    )(acc, den, b)


def _fin_mean_body(acc_ref, den_ref, b_ref, e_ref, x1_ref, x2_ref, o_ref):
    acc = jnp.concatenate([acc_ref[0], acc_ref[1]], axis=1)
    r = 1.0 / (den_ref[...] + 1e-16)
    o = acc * r[:, None] + b_ref[...]
    x3 = jnp.where(o > 0.0, o, jnp.exp(o) - 1.0)
    o_ref[...] = (e_ref[...] + x1_ref[...] + x2_ref[...] + x3) * 0.25


def _fin_mean(acc, den, b, emb, x1, x2):
    return pl.pallas_call(
        _fin_mean_body,
        grid=(NBLK,),
        in_specs=[
            pl.BlockSpec((2, ROWBLK, HALF), lambda i: (0, i, 0)),
            pl.BlockSpec((ROWBLK,), lambda i: (i,)),
            pl.BlockSpec((D,), lambda i: (0,)),
            pl.BlockSpec((ROWBLK, D), lambda i: (i, 0)),
            pl.BlockSpec((ROWBLK, D), lambda i: (i, 0)),
            pl.BlockSpec((ROWBLK, D), lambda i: (i, 0)),
        ],
        out_specs=pl.BlockSpec((ROWBLK, D), lambda i: (i, 0)),
        out_shape=jax.ShapeDtypeStruct((NPAD, D), jnp.float32),
    )(acc, den, b, emb, x1, x2)


def _edge_body(ed_hbm, h2_hbm, al_hbm, ar_hbm, acc0_hbm, den0_hbm,
               acco_hbm, deno_hbm,
               acc_sh, den_sh, alr_sh, tmp_v,
               rows, oh, sd, alv, arv, exv, ia, da, dr,
               gs, ss, es, asm):
    c = lax.axis_index("c")
    s = lax.axis_index("s")
    acc_rows_per_sub = NPAD // NS
    den_rows_per_sub = DENROWS // 8
    tab_per_sub = NPAD // NS

    @pl.loop(0, 28)
    def _init(j):
        off = s * acc_rows_per_sub + j * 112
        pltpu.sync_copy(acc0_hbm.at[c, pl.ds(off, 112)],
                        acc_sh.at[pl.ds(off, 112)])

    @pl.when(s < 8)
    def _():
        @pl.loop(0, 7)
        def _initd(j):
            off = s * den_rows_per_sub + j * 56
            pltpu.sync_copy(den0_hbm.at[pl.ds(off, 56)],
                            den_sh.at[pl.ds(off, 56)])

    for part, hbm in ((0, al_hbm), (1, ar_hbm)):
        @pl.loop(0, 8)
        def _tab(j, part=part, hbm=hbm):
            off = s * tab_per_sub + j * 392
            pltpu.sync_copy(hbm.at[pl.ds(off, 392)], tmp_v)
            pltpu.sync_copy(tmp_v, alr_sh.at[pl.ds(part * NPAD + off, 392)])
    plsc.subcore_barrier()

    coff = c * NPAD
    base0 = s * CHUNKS_PER_SUB

    def load_ed(t, b):
        return pltpu.async_copy(
            ed_hbm.at[pl.ds((base0 + t) * (2 * CHUNK), 2 * CHUNK)], sd[b],
            es[b])

    def wait_ed(t, b):
        pltpu.make_async_copy(
            ed_hbm.at[pl.ds((base0 + t) * (2 * CHUNK), 2 * CHUNK)], sd[b],
            es[b]).wait()

    def issue_alr(b):
        pltpu.async_copy(alr_sh.at[sd[b].at[pl.ds(0, CHUNK)]], alv[b], asm[b])
        pltpu.async_copy(alr_sh.at[sd[b].at[pl.ds(CHUNK, CHUNK)]], arv[b],
                         asm[b])

    def wait_alr(b):
        pltpu.make_async_copy(alr_sh.at[sd[b].at[pl.ds(0, CHUNK)]], alv[b],
                              asm[b]).wait()
        pltpu.make_async_copy(alr_sh.at[sd[b].at[pl.ds(CHUNK, CHUNK)]],
                              arv[b], asm[b]).wait()

    def compute_chunk(b):
        zero16 = jnp.zeros((LANES,), jnp.float32)
        for k in range(CHUNK // LANES):
            sl = pl.ds(k * LANES, LANES)
            s16 = sd[b][sl]
            dn16 = sd[b][pl.ds(CHUNK + k * LANES, LANES)]
            e = alv[b][sl] + arv[b][sl]
            e = jnp.where(e >= 0.0, e, e * 0.2)
            ex = jnp.exp(e)
            exv[b][sl] = ex
            d16 = dn16 - NPAD
            ia[b][sl] = coff + s16
            da[b][sl] = d16
            dr[b][sl] = lax.shift_right_logical(d16, 4)
            for j in range(LANES):
                oh[b].at[k * LANES + j][...] = zero16
            rowid = k * LANES + lax.iota(jnp.int32, LANES)
            plsc.store_scatter(oh[b], [rowid, d16 & 15], ex)

    def issue_gather(b):
        pltpu.async_copy(h2_hbm.at[ia[b]], rows[b], gs[b])

    def wait_gather(b):
        pltpu.make_async_copy(h2_hbm.at[ia[b]], rows[b], gs[b]).wait()

    def scale_chunk(b):
        @pl.loop(0, CHUNK, unroll=4)
        def _scale(r):
            bc = plsc.load_gather(exv[b], [jnp.full((LANES,), r, jnp.int32)])
            rows[b].at[r, pl.ds(0, LANES)][...] = (
                rows[b].at[r, pl.ds(0, LANES)][...] * bc)
            rows[b].at[r, pl.ds(LANES, LANES)][...] = (
                rows[b].at[r, pl.ds(LANES, LANES)][...] * bc)

    def issue_scatter(b):
        pltpu.async_copy(rows[b], acc_sh.at[da[b]], ss[b], add=True)
        pltpu.async_copy(oh[b], den_sh.at[dr[b]], ss[b], add=True)

    def wait_scatter(b):
        pltpu.make_async_copy(rows[b], acc_sh.at[da[b]], ss[b]).wait()
        pltpu.make_async_copy(oh[b], den_sh.at[dr[b]], ss[b]).wait()

    pltpu.sync_copy(ed_hbm.at[pl.ds(base0 * (2 * CHUNK), 2 * CHUNK)], sd[0])
    pltpu.sync_copy(alr_sh.at[sd[0].at[pl.ds(0, CHUNK)]], alv[0])
    pltpu.sync_copy(alr_sh.at[sd[0].at[pl.ds(CHUNK, CHUNK)]], arv[0])
    compute_chunk(0)
    issue_gather(0)
    load_ed(1, 1)

    @pl.loop(0, CHUNKS_PER_SUB // 2)
    def _g(i):
        for b in (0, 1):
            o = 1 - b
            t = 2 * i + b
            wait_gather(b)
            scale_chunk(b)
            issue_scatter(b)
            tn = t + 1

            @pl.when(tn < CHUNKS_PER_SUB)
            def _():
                wait_ed(tn, o)
                issue_alr(o)

                @pl.when(tn + 1 < CHUNKS_PER_SUB)
                def _():
                    load_ed(tn + 1, b)

                @pl.when(t >= 1)
                def _():
                    wait_scatter(o)
                wait_alr(o)
                compute_chunk(o)
                issue_gather(o)

    wait_scatter(0)
    wait_scatter(1)
    plsc.subcore_barrier()

    @pl.loop(0, 28)
    def _exp(j):
        off = s * acc_rows_per_sub + j * 112
        pltpu.sync_copy(acc_sh.at[pl.ds(off, 112)],
                        acco_hbm.at[c, pl.ds(off, 112)])

    @pl.when(s < 8)
    def _():
        @pl.loop(0, 7)
        def _expd(j):
            off = s * den_rows_per_sub + j * 56
            pltpu.sync_copy(den_sh.at[pl.ds(off, 56)],
                            deno_hbm.at[c, pl.ds(off, 56)])


def _edge(ed, h2, al, ar, acc0, den0):
    mesh = plsc.VectorSubcoreMesh(core_axis_name="c", subcore_axis_name="s",
                                  num_cores=NC, num_subcores=NS)
    cp = pltpu.CompilerParams(use_tc_tiling_on_sc=False)
    if "needs_layout_passes" in pltpu.CompilerParams.__dataclass_fields__:
        cp = dataclasses.replace(cp, needs_layout_passes=False)
    f = pl.kernel(
        _edge_body,
        out_type=[
            jax.ShapeDtypeStruct((2, NPAD, 2 * LANES), jnp.float32),
            jax.ShapeDtypeStruct((2, DENROWS, LANES), jnp.float32),
        ],
        mesh=mesh,
        scratch_types=[
            pltpu.VMEM_SHARED((NPAD, 2 * LANES), jnp.float32),
            pltpu.VMEM_SHARED((DENROWS, LANES), jnp.float32),
            pltpu.VMEM_SHARED((2 * NPAD,), jnp.float32),
            pltpu.VMEM((392,), jnp.float32),
            [pltpu.VMEM((CHUNK, 2 * LANES), jnp.float32) for _ in range(2)],
            [pltpu.VMEM((CHUNK, LANES), jnp.float32) for _ in range(2)],
            [pltpu.VMEM((2 * CHUNK,), jnp.int32) for _ in range(2)],
            [pltpu.VMEM((CHUNK,), jnp.float32) for _ in range(2)],
            [pltpu.VMEM((CHUNK,), jnp.float32) for _ in range(2)],
            [pltpu.VMEM((CHUNK,), jnp.float32) for _ in range(2)],
            [pltpu.VMEM((CHUNK,), jnp.int32) for _ in range(2)],
            [pltpu.VMEM((CHUNK,), jnp.int32) for _ in range(2)],
            [pltpu.VMEM((CHUNK,), jnp.int32) for _ in range(2)],
            [pltpu.SemaphoreType.DMA for _ in range(2)],
            [pltpu.SemaphoreType.DMA for _ in range(2)],
            [pltpu.SemaphoreType.DMA for _ in range(2)],
            [pltpu.SemaphoreType.DMA for _ in range(2)],
        ],
        compiler_params=cp,
    )
    return f(ed, h2, al, ar, acc0, den0)


def kernel(edge_index, emb, W0, a_src0, a_dst0, b0,
           W1, a_src1, a_dst1, b1, W2, a_src2, a_dst2, b2):
    ei = edge_index.astype(jnp.int32)
    pad_nodes = N + (jnp.arange(EPAD - E, dtype=jnp.int32) % NPADROWS)
    srci = jnp.concatenate([ei[0], pad_nodes])
    dsti = jnp.concatenate([ei[1], pad_nodes])
    ed = jnp.stack([srci.reshape(-1, CHUNK),
                    (dsti + NPAD).reshape(-1, CHUNK)], axis=1).reshape(-1)

    emb_pad = jnp.zeros((NPAD, D), jnp.float32).at[:N].set(emb)

    x = emb_pad
    hist = []
    out = None
    for li, (w, a_s, a_d, b) in enumerate(
            ((W0, a_src0, a_dst0, b0), (W1, a_src1, a_dst1, b1),
             (W2, a_src2, a_dst2, b2))):
        h_all, acc_all, al, ar, den0 = _prep(x, w, a_s, a_d)
        h2 = h_all.reshape(2 * NPAD, HALF)
        acco, deno = _edge(ed, h2, al, ar, acc_all,
                           den0.reshape(DENROWS, LANES))
        accr = acco
        den = deno[0].reshape(NPAD)
        if li < 2:
            x = _fin(accr, den, b)
            hist.append(x)
        else:
            out = _fin_mean(accr, den, b, emb_pad, hist[0], hist[1])
    return out[:N]


# speedup vs baseline: 27.7982x; 1.1289x over previous
import dataclasses

import jax
import jax.numpy as jnp
from jax import lax
from jax.experimental import pallas as pl
from jax.experimental.pallas import tpu as pltpu
from jax.experimental.pallas import tpu_sc as plsc

N = 50000
D = 64
E = 800000

LANES = 16
NC = 2
NS = 16
HALF = D // NC

ROWBLK = 512
NPAD = 50176
NBLK = NPAD // ROWBLK

CHUNK = 128
EPAD = 819200
CHUNKS_PER_SUB = EPAD // (CHUNK * NS)
NPADROWS = 48
DENROWS = NPAD // LANES
NROWS = NPAD * 2 + DENROWS


def _prep_body(x_ref, w_ref, asr_ref, adr_ref,
               h_ref, acc_ref, al_ref, ar_ref, den_ref):
    x = x_ref[...]
    h = jnp.dot(x, w_ref[...], preferred_element_type=jnp.float32)
    al = jnp.sum(h * asr_ref[...], axis=1)
    ar = jnp.sum(h * adr_ref[...], axis=1)
    e = al + ar
    e = jnp.where(e >= 0.0, e, e * 0.2)
    exs = jnp.exp(e)
    h_ref[0, ...] = h[:, :HALF]
    h_ref[1, ...] = h[:, HALF:]
    acc = h * exs[:, None]
    acc_ref[0, ...] = acc[:, :HALF]
    acc_ref[1, ...] = acc[:, HALF:]
    al_ref[...] = al
    ar_ref[...] = ar
    den_ref[...] = exs


def _prep(x, w, a_s, a_d):
    return pl.pallas_call(
        _prep_body,
        grid=(NBLK,),
        in_specs=[
            pl.BlockSpec((ROWBLK, D), lambda i: (i, 0)),
            pl.BlockSpec((D, D), lambda i: (0, 0)),
            pl.BlockSpec((D,), lambda i: (0,)),
            pl.BlockSpec((D,), lambda i: (0,)),
        ],
        out_specs=[
            pl.BlockSpec((2, ROWBLK, HALF), lambda i: (0, i, 0)),
            pl.BlockSpec((2, ROWBLK, HALF), lambda i: (0, i, 0)),
            pl.BlockSpec((ROWBLK,), lambda i: (i,)),
            pl.BlockSpec((ROWBLK,), lambda i: (i,)),
            pl.BlockSpec((ROWBLK,), lambda i: (i,)),
        ],
        out_shape=[
            jax.ShapeDtypeStruct((2, NPAD, HALF), jnp.float32),
            jax.ShapeDtypeStruct((2, NPAD, HALF), jnp.float32),
            jax.ShapeDtypeStruct((NPAD,), jnp.float32),
            jax.ShapeDtypeStruct((NPAD,), jnp.float32),
            jax.ShapeDtypeStruct((NPAD,), jnp.float32),
        ],
    )(x, w, a_s, a_d)


def _step_body(acc_ref, den_ref, bp_ref, w_ref, asr_ref, adr_ref,
               x_ref, h_ref, accn_ref, al_ref, ar_ref, den_out_ref):
    acc = jnp.concatenate([acc_ref[0], acc_ref[1]], axis=1)
    r = 1.0 / (den_ref[...] + 1e-16)
    o = acc * r[:, None] + bp_ref[...]
    x = jnp.where(o > 0.0, o, jnp.exp(o) - 1.0)
    x_ref[...] = x
    h = jnp.dot(x, w_ref[...], preferred_element_type=jnp.float32)
    al = jnp.sum(h * asr_ref[...], axis=1)
    ar = jnp.sum(h * adr_ref[...], axis=1)
    e = al + ar
    e = jnp.where(e >= 0.0, e, e * 0.2)
    exs = jnp.exp(e)
    h_ref[0, ...] = h[:, :HALF]
    h_ref[1, ...] = h[:, HALF:]
    accn = h * exs[:, None]
    accn_ref[0, ...] = accn[:, :HALF]
    accn_ref[1, ...] = accn[:, HALF:]
    al_ref[...] = al
    ar_ref[...] = ar
    den_out_ref[...] = exs


def _step(acc, den, bp, w, a_s, a_d):
    return pl.pallas_call(
        _step_body,
        grid=(NBLK,),
        in_specs=[
            pl.BlockSpec((2, ROWBLK, HALF), lambda i: (0, i, 0)),
            pl.BlockSpec((ROWBLK,), lambda i: (i,)),
            pl.BlockSpec((D,), lambda i: (0,)),
            pl.BlockSpec((D, D), lambda i: (0, 0)),
            pl.BlockSpec((D,), lambda i: (0,)),
            pl.BlockSpec((D,), lambda i: (0,)),
        ],
        out_specs=[
            pl.BlockSpec((ROWBLK, D), lambda i: (i, 0)),
            pl.BlockSpec((2, ROWBLK, HALF), lambda i: (0, i, 0)),
            pl.BlockSpec((2, ROWBLK, HALF), lambda i: (0, i, 0)),
            pl.BlockSpec((ROWBLK,), lambda i: (i,)),
            pl.BlockSpec((ROWBLK,), lambda i: (i,)),
            pl.BlockSpec((ROWBLK,), lambda i: (i,)),
        ],
        out_shape=[
            jax.ShapeDtypeStruct((NPAD, D), jnp.float32),
            jax.ShapeDtypeStruct((2, NPAD, HALF), jnp.float32),
            jax.ShapeDtypeStruct((2, NPAD, HALF), jnp.float32),
            jax.ShapeDtypeStruct((NPAD,), jnp.float32),
            jax.ShapeDtypeStruct((NPAD,), jnp.float32),
            jax.ShapeDtypeStruct((NPAD,), jnp.float32),
        ],
    )(acc, den, bp, w, a_s, a_d)


def _fin_mean_body(acc_ref, den_ref, b_ref, e_ref, x1_ref, x2_ref, o_ref):
    acc = jnp.concatenate([acc_ref[0], acc_ref[1]], axis=1)
    r = 1.0 / (den_ref[...] + 1e-16)
    o = acc * r[:, None] + b_ref[...]
    x3 = jnp.where(o > 0.0, o, jnp.exp(o) - 1.0)
    o_ref[...] = (e_ref[...] + x1_ref[...] + x2_ref[...] + x3) * 0.25


def _fin_mean(acc, den, b, emb, x1, x2):
    return pl.pallas_call(
        _fin_mean_body,
        grid=(NBLK,),
        in_specs=[
            pl.BlockSpec((2, ROWBLK, HALF), lambda i: (0, i, 0)),
            pl.BlockSpec((ROWBLK,), lambda i: (i,)),
            pl.BlockSpec((D,), lambda i: (0,)),
            pl.BlockSpec((ROWBLK, D), lambda i: (i, 0)),
            pl.BlockSpec((ROWBLK, D), lambda i: (i, 0)),
            pl.BlockSpec((ROWBLK, D), lambda i: (i, 0)),
        ],
        out_specs=pl.BlockSpec((ROWBLK, D), lambda i: (i, 0)),
        out_shape=jax.ShapeDtypeStruct((NPAD, D), jnp.float32),
    )(acc, den, b, emb, x1, x2)


def _edge_body(ed_hbm, h2_hbm, al_hbm, ar_hbm, acc0_hbm, den0_hbm,
               acco_hbm, deno_hbm,
               acc_sh, den_sh, alr_sh, tmp_v,
               rows, oh, sd, alv, arv, exv, ia, da, dr,
               gs, ss, es, asm):
    c = lax.axis_index("c")
    s = lax.axis_index("s")
    acc_rows_per_sub = NPAD // NS
    den_rows_per_sub = DENROWS // 8
    tab_per_sub = NPAD // NS

    @pl.loop(0, 28)
    def _init(j):
        off = s * acc_rows_per_sub + j * 112
        pltpu.sync_copy(acc0_hbm.at[c, pl.ds(off, 112)],
                        acc_sh.at[pl.ds(off, 112)])

    @pl.when(s < 8)
    def _():
        @pl.loop(0, 7)
        def _initd(j):
            off = s * den_rows_per_sub + j * 56
            pltpu.sync_copy(den0_hbm.at[pl.ds(off, 56)],
                            den_sh.at[pl.ds(off, 56)])

    for part, hbm in ((0, al_hbm), (1, ar_hbm)):
        @pl.loop(0, 8)
        def _tab(j, part=part, hbm=hbm):
            off = s * tab_per_sub + j * 392
            pltpu.sync_copy(hbm.at[pl.ds(off, 392)], tmp_v)
            pltpu.sync_copy(tmp_v, alr_sh.at[pl.ds(part * NPAD + off, 392)])
    plsc.subcore_barrier()

    coff = c * NPAD
    base0 = s * CHUNKS_PER_SUB

    def load_ed(t, b):
        return pltpu.async_copy(
            ed_hbm.at[pl.ds((base0 + t) * (2 * CHUNK), 2 * CHUNK)], sd[b],
            es[b])

    def wait_ed(t, b):
        pltpu.make_async_copy(
            ed_hbm.at[pl.ds((base0 + t) * (2 * CHUNK), 2 * CHUNK)], sd[b],
            es[b]).wait()

    def issue_alr(b):
        pltpu.async_copy(alr_sh.at[sd[b].at[pl.ds(0, CHUNK)]], alv[b], asm[b])
        pltpu.async_copy(alr_sh.at[sd[b].at[pl.ds(CHUNK, CHUNK)]], arv[b],
                         asm[b])

    def wait_alr(b):
        pltpu.make_async_copy(alr_sh.at[sd[b].at[pl.ds(0, CHUNK)]], alv[b],
                              asm[b]).wait()
        pltpu.make_async_copy(alr_sh.at[sd[b].at[pl.ds(CHUNK, CHUNK)]],
                              arv[b], asm[b]).wait()

    def compute_chunk(b):
        zero16 = jnp.zeros((LANES,), jnp.float32)
        for k in range(CHUNK // LANES):
            sl = pl.ds(k * LANES, LANES)
            s16 = sd[b][sl]
            dn16 = sd[b][pl.ds(CHUNK + k * LANES, LANES)]
            e = alv[b][sl] + arv[b][sl]
            e = jnp.where(e >= 0.0, e, e * 0.2)
            ex = jnp.exp(e)
            exv[b][sl] = ex
            d16 = dn16 - NPAD
            ia[b][sl] = coff + s16
            da[b][sl] = d16
            dr[b][sl] = lax.shift_right_logical(d16, 4)
            for j in range(LANES):
                oh[b].at[k * LANES + j][...] = zero16
            rowid = k * LANES + lax.iota(jnp.int32, LANES)
            plsc.store_scatter(oh[b], [rowid, d16 & 15], ex)

    def issue_gather(b):
        pltpu.async_copy(h2_hbm.at[ia[b]], rows[b], gs[b])

    def wait_gather(b):
        pltpu.make_async_copy(h2_hbm.at[ia[b]], rows[b], gs[b]).wait()

    def scale_chunk(b):
        @pl.loop(0, CHUNK, unroll=8)
        def _scale(r):
            bc = plsc.load_gather(exv[b], [jnp.full((LANES,), r, jnp.int32)])
            rows[b].at[r, pl.ds(0, LANES)][...] = (
                rows[b].at[r, pl.ds(0, LANES)][...] * bc)
            rows[b].at[r, pl.ds(LANES, LANES)][...] = (
                rows[b].at[r, pl.ds(LANES, LANES)][...] * bc)

    def issue_scatter(b):
        pltpu.async_copy(rows[b], acc_sh.at[da[b]], ss[b], add=True)
        pltpu.async_copy(oh[b], den_sh.at[dr[b]], ss[b], add=True)

    def wait_scatter(b):
        pltpu.make_async_copy(rows[b], acc_sh.at[da[b]], ss[b]).wait()
        pltpu.make_async_copy(oh[b], den_sh.at[dr[b]], ss[b]).wait()

    pltpu.sync_copy(ed_hbm.at[pl.ds(base0 * (2 * CHUNK), 2 * CHUNK)], sd[0])
    pltpu.sync_copy(alr_sh.at[sd[0].at[pl.ds(0, CHUNK)]], alv[0])
    pltpu.sync_copy(alr_sh.at[sd[0].at[pl.ds(CHUNK, CHUNK)]], arv[0])
    compute_chunk(0)
    issue_gather(0)
    load_ed(1, 1)

    @pl.loop(0, CHUNKS_PER_SUB // 2)
    def _g(i):
        for b in (0, 1):
            o = 1 - b
            t = 2 * i + b
            wait_gather(b)
            scale_chunk(b)
            issue_scatter(b)
            tn = t + 1

            @pl.when(tn < CHUNKS_PER_SUB)
            def _():
                wait_ed(tn, o)
                issue_alr(o)

                @pl.when(tn + 1 < CHUNKS_PER_SUB)
                def _():
                    load_ed(tn + 1, b)

                @pl.when(t >= 1)
                def _():
                    wait_scatter(o)
                wait_alr(o)
                compute_chunk(o)
                issue_gather(o)

    wait_scatter(0)
    wait_scatter(1)
    plsc.subcore_barrier()

    @pl.loop(0, 28)
    def _exp(j):
        off = s * acc_rows_per_sub + j * 112
        pltpu.sync_copy(acc_sh.at[pl.ds(off, 112)],
                        acco_hbm.at[c, pl.ds(off, 112)])

    @pl.when(s < 8)
    def _():
        @pl.loop(0, 7)
        def _expd(j):
            off = s * den_rows_per_sub + j * 56
            pltpu.sync_copy(den_sh.at[pl.ds(off, 56)],
                            deno_hbm.at[c, pl.ds(off, 56)])


def _edge(ed, h2, al, ar, acc0, den0):
    mesh = plsc.VectorSubcoreMesh(core_axis_name="c", subcore_axis_name="s",
                                  num_cores=NC, num_subcores=NS)
    cp = pltpu.CompilerParams(use_tc_tiling_on_sc=False)
    if "needs_layout_passes" in pltpu.CompilerParams.__dataclass_fields__:
        cp = dataclasses.replace(cp, needs_layout_passes=False)
    f = pl.kernel(
        _edge_body,
        out_type=[
            jax.ShapeDtypeStruct((2, NPAD, 2 * LANES), jnp.float32),
            jax.ShapeDtypeStruct((2, DENROWS, LANES), jnp.float32),
        ],
        mesh=mesh,
        scratch_types=[
            pltpu.VMEM_SHARED((NPAD, 2 * LANES), jnp.float32),
            pltpu.VMEM_SHARED((DENROWS, LANES), jnp.float32),
            pltpu.VMEM_SHARED((2 * NPAD,), jnp.float32),
            pltpu.VMEM((392,), jnp.float32),
            [pltpu.VMEM((CHUNK, 2 * LANES), jnp.float32) for _ in range(2)],
            [pltpu.VMEM((CHUNK, LANES), jnp.float32) for _ in range(2)],
            [pltpu.VMEM((2 * CHUNK,), jnp.int32) for _ in range(2)],
            [pltpu.VMEM((CHUNK,), jnp.float32) for _ in range(2)],
            [pltpu.VMEM((CHUNK,), jnp.float32) for _ in range(2)],
            [pltpu.VMEM((CHUNK,), jnp.float32) for _ in range(2)],
            [pltpu.VMEM((CHUNK,), jnp.int32) for _ in range(2)],
            [pltpu.VMEM((CHUNK,), jnp.int32) for _ in range(2)],
            [pltpu.VMEM((CHUNK,), jnp.int32) for _ in range(2)],
            [pltpu.SemaphoreType.DMA for _ in range(2)],
            [pltpu.SemaphoreType.DMA for _ in range(2)],
            [pltpu.SemaphoreType.DMA for _ in range(2)],
            [pltpu.SemaphoreType.DMA for _ in range(2)],
        ],
        compiler_params=cp,
    )
    return f(ed, h2, al, ar, acc0, den0)


def kernel(edge_index, emb, W0, a_src0, a_dst0, b0,
           W1, a_src1, a_dst1, b1, W2, a_src2, a_dst2, b2):
    ei = edge_index.astype(jnp.int32)
    pad_nodes = N + (jnp.arange(EPAD - E, dtype=jnp.int32) % NPADROWS)
    srci = jnp.concatenate([ei[0], pad_nodes])
    dsti = jnp.concatenate([ei[1], pad_nodes])
    ed = jnp.stack([srci.reshape(-1, CHUNK),
                    (dsti + NPAD).reshape(-1, CHUNK)], axis=1).reshape(-1)

    emb_pad = jnp.zeros((NPAD, D), jnp.float32).at[:N].set(emb)

    h_all, acc_all, al, ar, den0 = _prep(emb_pad, W0, a_src0, a_dst0)
    xs = []
    for li, (bp, w, a_s, a_d) in enumerate(
            ((b0, W1, a_src1, a_dst1), (b1, W2, a_src2, a_dst2))):
        h2 = h_all.reshape(2 * NPAD, HALF)
        acco, deno = _edge(ed, h2, al, ar, acc_all,
                           den0.reshape(DENROWS, LANES))
        den = deno[0].reshape(NPAD)
        x, h_all, acc_all, al, ar, den0 = _step(acco, den, bp, w, a_s, a_d)
        xs.append(x)
    h2 = h_all.reshape(2 * NPAD, HALF)
    acco, deno = _edge(ed, h2, al, ar, acc_all, den0.reshape(DENROWS, LANES))
    den = deno[0].reshape(NPAD)
    out = _fin_mean(acco, den, b2, emb_pad, xs[0], xs[1])
    return out[:N]


# speedup vs baseline: 35.5599x; 1.2792x over previous
import dataclasses

import jax
import jax.numpy as jnp
from jax import lax
from jax.experimental import pallas as pl
from jax.experimental.pallas import tpu as pltpu
from jax.experimental.pallas import tpu_sc as plsc

N = 50000
D = 64
E = 800000

LANES = 16
NC = 2
NS = 16
HALF = D // NC

ROWBLK = 512
NPAD = 50176
NBLK = NPAD // ROWBLK

CHUNK = 128
EPAD = 819200
CHUNKS_PER_SUB = EPAD // (CHUNK * NS)
NPADROWS = 48
DENROWS = NPAD // LANES
NROWS = NPAD * 2 + DENROWS


def _prep_body(x_ref, w_ref, asr_ref, adr_ref,
               h_ref, acc_ref, al_ref, ar_ref, den_ref):
    x = x_ref[...]
    h = jnp.dot(x, w_ref[...], preferred_element_type=jnp.float32)
    al = jnp.sum(h * asr_ref[...], axis=1)
    ar = jnp.sum(h * adr_ref[...], axis=1)
    e = al + ar
    e = jnp.where(e >= 0.0, e, e * 0.2)
    exs = jnp.exp(e)
    h_ref[0, ...] = h[:, :HALF]
    h_ref[1, ...] = h[:, HALF:]
    acc = h * exs[:, None]
    acc_ref[0, ...] = acc[:, :HALF]
    acc_ref[1, ...] = acc[:, HALF:]
    al_ref[...] = al
    ar_ref[...] = ar
    den_ref[...] = exs


def _prep(x, w, a_s, a_d):
    return pl.pallas_call(
        _prep_body,
        grid=(NBLK,),
        in_specs=[
            pl.BlockSpec((ROWBLK, D), lambda i: (i, 0)),
            pl.BlockSpec((D, D), lambda i: (0, 0)),
            pl.BlockSpec((D,), lambda i: (0,)),
            pl.BlockSpec((D,), lambda i: (0,)),
        ],
        out_specs=[
            pl.BlockSpec((2, ROWBLK, HALF), lambda i: (0, i, 0)),
            pl.BlockSpec((2, ROWBLK, HALF), lambda i: (0, i, 0)),
            pl.BlockSpec((ROWBLK,), lambda i: (i,)),
            pl.BlockSpec((ROWBLK,), lambda i: (i,)),
            pl.BlockSpec((ROWBLK,), lambda i: (i,)),
        ],
        out_shape=[
            jax.ShapeDtypeStruct((2, NPAD, HALF), jnp.float32),
            jax.ShapeDtypeStruct((2, NPAD, HALF), jnp.float32),
            jax.ShapeDtypeStruct((NPAD,), jnp.float32),
            jax.ShapeDtypeStruct((NPAD,), jnp.float32),
            jax.ShapeDtypeStruct((NPAD,), jnp.float32),
        ],
    )(x, w, a_s, a_d)


def _step_body(acc_ref, den_ref, bp_ref, w_ref, asr_ref, adr_ref,
               x_ref, h_ref, accn_ref, al_ref, ar_ref, den_out_ref):
    acc = jnp.concatenate([acc_ref[0], acc_ref[1]], axis=1)
    r = 1.0 / (den_ref[...] + 1e-16)
    o = acc * r[:, None] + bp_ref[...]
    x = jnp.where(o > 0.0, o, jnp.exp(o) - 1.0)
    x_ref[...] = x
    h = jnp.dot(x, w_ref[...], preferred_element_type=jnp.float32)
    al = jnp.sum(h * asr_ref[...], axis=1)
    ar = jnp.sum(h * adr_ref[...], axis=1)
    e = al + ar
    e = jnp.where(e >= 0.0, e, e * 0.2)
    exs = jnp.exp(e)
    h_ref[0, ...] = h[:, :HALF]
    h_ref[1, ...] = h[:, HALF:]
    accn = h * exs[:, None]
    accn_ref[0, ...] = accn[:, :HALF]
    accn_ref[1, ...] = accn[:, HALF:]
    al_ref[...] = al
    ar_ref[...] = ar
    den_out_ref[...] = exs


def _step(acc, den, bp, w, a_s, a_d):
    return pl.pallas_call(
        _step_body,
        grid=(NBLK,),
        in_specs=[
            pl.BlockSpec((2, ROWBLK, HALF), lambda i: (0, i, 0)),
            pl.BlockSpec((ROWBLK,), lambda i: (i,)),
            pl.BlockSpec((D,), lambda i: (0,)),
            pl.BlockSpec((D, D), lambda i: (0, 0)),
            pl.BlockSpec((D,), lambda i: (0,)),
            pl.BlockSpec((D,), lambda i: (0,)),
        ],
        out_specs=[
            pl.BlockSpec((ROWBLK, D), lambda i: (i, 0)),
            pl.BlockSpec((2, ROWBLK, HALF), lambda i: (0, i, 0)),
            pl.BlockSpec((2, ROWBLK, HALF), lambda i: (0, i, 0)),
            pl.BlockSpec((ROWBLK,), lambda i: (i,)),
            pl.BlockSpec((ROWBLK,), lambda i: (i,)),
            pl.BlockSpec((ROWBLK,), lambda i: (i,)),
        ],
        out_shape=[
            jax.ShapeDtypeStruct((NPAD, D), jnp.float32),
            jax.ShapeDtypeStruct((2, NPAD, HALF), jnp.float32),
            jax.ShapeDtypeStruct((2, NPAD, HALF), jnp.float32),
            jax.ShapeDtypeStruct((NPAD,), jnp.float32),
            jax.ShapeDtypeStruct((NPAD,), jnp.float32),
            jax.ShapeDtypeStruct((NPAD,), jnp.float32),
        ],
    )(acc, den, bp, w, a_s, a_d)


def _fin_mean_body(acc_ref, den_ref, b_ref, e_ref, x1_ref, x2_ref, o_ref):
    acc = jnp.concatenate([acc_ref[0], acc_ref[1]], axis=1)
    r = 1.0 / (den_ref[...] + 1e-16)
    o = acc * r[:, None] + b_ref[...]
    x3 = jnp.where(o > 0.0, o, jnp.exp(o) - 1.0)
    o_ref[...] = (e_ref[...] + x1_ref[...] + x2_ref[...] + x3) * 0.25


def _fin_mean(acc, den, b, emb, x1, x2):
    return pl.pallas_call(
        _fin_mean_body,
        grid=(NBLK,),
        in_specs=[
            pl.BlockSpec((2, ROWBLK, HALF), lambda i: (0, i, 0)),
            pl.BlockSpec((ROWBLK,), lambda i: (i,)),
            pl.BlockSpec((D,), lambda i: (0,)),
            pl.BlockSpec((ROWBLK, D), lambda i: (i, 0)),
            pl.BlockSpec((ROWBLK, D), lambda i: (i, 0)),
            pl.BlockSpec((ROWBLK, D), lambda i: (i, 0)),
        ],
        out_specs=pl.BlockSpec((ROWBLK, D), lambda i: (i, 0)),
        out_shape=jax.ShapeDtypeStruct((NPAD, D), jnp.float32),
    )(acc, den, b, emb, x1, x2)


def _edge_body(ed_hbm, h2_hbm, al_hbm, ar_hbm, acc0_hbm, den0_hbm,
               acco_hbm, deno_hbm,
               acc_sh, den_sh, alr_sh, tmp_v,
               rows, oh, sd, alv, arv, exv, ia, da, dr,
               gs, ss, es, asm):
    c = lax.axis_index("c")
    s = lax.axis_index("s")
    acc_rows_per_sub = NPAD // NS
    den_rows_per_sub = DENROWS // 8
    tab_per_sub = NPAD // NS

    @pl.loop(0, 28)
    def _init(j):
        off = s * acc_rows_per_sub + j * 112
        pltpu.sync_copy(acc0_hbm.at[c, pl.ds(off, 112)],
                        acc_sh.at[pl.ds(off, 112)])

    @pl.when(s < 8)
    def _():
        @pl.loop(0, 7)
        def _initd(j):
            off = s * den_rows_per_sub + j * 56
            pltpu.sync_copy(den0_hbm.at[pl.ds(off, 56)],
                            den_sh.at[pl.ds(off, 56)])

    for part, hbm in ((0, al_hbm), (1, ar_hbm)):
        @pl.loop(0, 8)
        def _tab(j, part=part, hbm=hbm):
            off = s * tab_per_sub + j * 392
            pltpu.sync_copy(hbm.at[pl.ds(off, 392)], tmp_v)
            pltpu.sync_copy(tmp_v, alr_sh.at[pl.ds(part * NPAD + off, 392)])
    plsc.subcore_barrier()

    coff = c * NPAD
    base0 = s * CHUNKS_PER_SUB

    def load_ed(t, b):
        return pltpu.async_copy(
            ed_hbm.at[pl.ds((base0 + t) * (2 * CHUNK), 2 * CHUNK)], sd[b],
            es[b])

    def wait_ed(t, b):
        pltpu.make_async_copy(
            ed_hbm.at[pl.ds((base0 + t) * (2 * CHUNK), 2 * CHUNK)], sd[b],
            es[b]).wait()

    def issue_alr(b):
        pltpu.async_copy(alr_sh.at[sd[b].at[pl.ds(0, CHUNK)]], alv[b], asm[b])
        pltpu.async_copy(alr_sh.at[sd[b].at[pl.ds(CHUNK, CHUNK)]], arv[b],
                         asm[b])

    def wait_alr(b):
        pltpu.make_async_copy(alr_sh.at[sd[b].at[pl.ds(0, CHUNK)]], alv[b],
                              asm[b]).wait()
        pltpu.make_async_copy(alr_sh.at[sd[b].at[pl.ds(CHUNK, CHUNK)]],
                              arv[b], asm[b]).wait()

    def compute_chunk(b):
        zero16 = jnp.zeros((LANES,), jnp.float32)
        for k in range(CHUNK // LANES):
            sl = pl.ds(k * LANES, LANES)
            s16 = sd[b][sl]
            dn16 = sd[b][pl.ds(CHUNK + k * LANES, LANES)]
            e = alv[b][sl] + arv[b][sl]
            e = jnp.where(e >= 0.0, e, e * 0.2)
            ex = jnp.exp(e)
            exv[b][sl] = ex
            d16 = dn16 - NPAD
            ia[b][sl] = coff + s16
            da[b][sl] = d16
            dr[b][sl] = lax.shift_right_logical(d16, 4)
            for j in range(LANES):
                oh[b].at[k * LANES + j][...] = zero16
            rowid = k * LANES + lax.iota(jnp.int32, LANES)
            plsc.store_scatter(oh[b], [rowid, d16 & 15], ex)

    def issue_gather(b):
        pltpu.async_copy(h2_hbm.at[ia[b]], rows[b], gs[b])

    def wait_gather(b):
        pltpu.make_async_copy(h2_hbm.at[ia[b]], rows[b], gs[b]).wait()

    def scale_chunk(b):
        @pl.loop(0, CHUNK, unroll=8)
        def _scale(r):
            bc = plsc.load_gather(exv[b], [jnp.full((LANES,), r, jnp.int32)])
            rows[b].at[r, pl.ds(0, LANES)][...] = (
                rows[b].at[r, pl.ds(0, LANES)][...] * bc)
            rows[b].at[r, pl.ds(LANES, LANES)][...] = (
                rows[b].at[r, pl.ds(LANES, LANES)][...] * bc)

    def issue_scatter(b):
        pltpu.async_copy(rows[b], acc_sh.at[da[b]], ss[b], add=True)
        pltpu.async_copy(oh[b], den_sh.at[dr[b]], ss[b], add=True)

    def wait_scatter(b):
        pltpu.make_async_copy(rows[b], acc_sh.at[da[b]], ss[b]).wait()
        pltpu.make_async_copy(oh[b], den_sh.at[dr[b]], ss[b]).wait()

    pltpu.sync_copy(ed_hbm.at[pl.ds(base0 * (2 * CHUNK), 2 * CHUNK)], sd[0])
    pltpu.sync_copy(alr_sh.at[sd[0].at[pl.ds(0, CHUNK)]], alv[0])
    pltpu.sync_copy(alr_sh.at[sd[0].at[pl.ds(CHUNK, CHUNK)]], arv[0])
    compute_chunk(0)
    issue_gather(0)
    load_ed(1, 1)

    @pl.loop(0, CHUNKS_PER_SUB // 2)
    def _g(i):
        for b in (0, 1):
            o = 1 - b
            t = 2 * i + b
            tn = t + 1

            @pl.when(tn < CHUNKS_PER_SUB)
            def _():
                wait_ed(tn, o)
                issue_alr(o)

                @pl.when(tn + 1 < CHUNKS_PER_SUB)
                def _():
                    load_ed(tn + 1, b)

                @pl.when(t >= 1)
                def _():
                    wait_scatter(o)
                wait_alr(o)
                compute_chunk(o)
                issue_gather(o)

            wait_gather(b)
            scale_chunk(b)
            issue_scatter(b)

    wait_scatter(0)
    wait_scatter(1)
    plsc.subcore_barrier()

    @pl.loop(0, 28)
    def _exp(j):
        off = s * acc_rows_per_sub + j * 112
        pltpu.sync_copy(acc_sh.at[pl.ds(off, 112)],
                        acco_hbm.at[c, pl.ds(off, 112)])

    @pl.when(s < 8)
    def _():
        @pl.loop(0, 7)
        def _expd(j):
            off = s * den_rows_per_sub + j * 56
            pltpu.sync_copy(den_sh.at[pl.ds(off, 56)],
                            deno_hbm.at[c, pl.ds(off, 56)])


def _edge(ed, h2, al, ar, acc0, den0):
    mesh = plsc.VectorSubcoreMesh(core_axis_name="c", subcore_axis_name="s",
                                  num_cores=NC, num_subcores=NS)
    cp = pltpu.CompilerParams(use_tc_tiling_on_sc=False)
    if "needs_layout_passes" in pltpu.CompilerParams.__dataclass_fields__:
        cp = dataclasses.replace(cp, needs_layout_passes=False)
    f = pl.kernel(
        _edge_body,
        out_type=[
            jax.ShapeDtypeStruct((2, NPAD, 2 * LANES), jnp.float32),
            jax.ShapeDtypeStruct((2, DENROWS, LANES), jnp.float32),
        ],
        mesh=mesh,
        scratch_types=[
            pltpu.VMEM_SHARED((NPAD, 2 * LANES), jnp.float32),
            pltpu.VMEM_SHARED((DENROWS, LANES), jnp.float32),
            pltpu.VMEM_SHARED((2 * NPAD,), jnp.float32),
            pltpu.VMEM((392,), jnp.float32),
            [pltpu.VMEM((CHUNK, 2 * LANES), jnp.float32) for _ in range(2)],
            [pltpu.VMEM((CHUNK, LANES), jnp.float32) for _ in range(2)],
            [pltpu.VMEM((2 * CHUNK,), jnp.int32) for _ in range(2)],
            [pltpu.VMEM((CHUNK,), jnp.float32) for _ in range(2)],
            [pltpu.VMEM((CHUNK,), jnp.float32) for _ in range(2)],
            [pltpu.VMEM((CHUNK,), jnp.float32) for _ in range(2)],
            [pltpu.VMEM((CHUNK,), jnp.int32) for _ in range(2)],
            [pltpu.VMEM((CHUNK,), jnp.int32) for _ in range(2)],
            [pltpu.VMEM((CHUNK,), jnp.int32) for _ in range(2)],
            [pltpu.SemaphoreType.DMA for _ in range(2)],
            [pltpu.SemaphoreType.DMA for _ in range(2)],
            [pltpu.SemaphoreType.DMA for _ in range(2)],
            [pltpu.SemaphoreType.DMA for _ in range(2)],
        ],
        compiler_params=cp,
    )
    return f(ed, h2, al, ar, acc0, den0)


def kernel(edge_index, emb, W0, a_src0, a_dst0, b0,
           W1, a_src1, a_dst1, b1, W2, a_src2, a_dst2, b2):
    ei = edge_index.astype(jnp.int32)
    pad_nodes = N + (jnp.arange(EPAD - E, dtype=jnp.int32) % NPADROWS)
    srci = jnp.concatenate([ei[0], pad_nodes])
    dsti = jnp.concatenate([ei[1], pad_nodes])
    ed = jnp.stack([srci.reshape(-1, CHUNK),
                    (dsti + NPAD).reshape(-1, CHUNK)], axis=1).reshape(-1)

    emb_pad = jnp.zeros((NPAD, D), jnp.float32).at[:N].set(emb)

    h_all, acc_all, al, ar, den0 = _prep(emb_pad, W0, a_src0, a_dst0)
    xs = []
    for li, (bp, w, a_s, a_d) in enumerate(
            ((b0, W1, a_src1, a_dst1), (b1, W2, a_src2, a_dst2))):
        h2 = h_all.reshape(2 * NPAD, HALF)
        acco, deno = _edge(ed, h2, al, ar, acc_all,
                           den0.reshape(DENROWS, LANES))
        den = deno[0].reshape(NPAD)
        x, h_all, acc_all, al, ar, den0 = _step(acco, den, bp, w, a_s, a_d)
        xs.append(x)
    h2 = h_all.reshape(2 * NPAD, HALF)
    acco, deno = _edge(ed, h2, al, ar, acc_all, den0.reshape(DENROWS, LANES))
    den = deno[0].reshape(NPAD)
    out = _fin_mean(acco, den, b2, emb_pad, xs[0], xs[1])
    return out[:N]
